# Initial kernel scaffold; baseline (speedup 1.0000x reference)
import jax
import jax.numpy as jnp
from jax.experimental import pallas as pl


def kernel(source_position, target_position, source_values):
    raise NotImplementedError("write your pallas kernel here")


# speedup vs baseline: 1.3514x; 1.3514x over previous
import functools

import jax
import jax.numpy as jnp
from jax import lax
from jax.experimental import pallas as pl
from jax.experimental.pallas import tpu as pltpu
from jax.experimental.pallas import tpu_sc as plsc

T = 16384
S = 16384
D = 128
KNN = 9
LVL = 4
TB = 256
RG = 8
SLAB = 1024
NQ = S // SLAB
NCH = 128
KP = 12
WP = 16
INVL2 = 100.0
EPS = 1e-5
BIGF = 3.0e38


def _bf16_rne(x):
    b = jax.lax.bitcast_convert_type(x, jnp.uint32)
    b = b + jnp.uint32(0x7FFF) + ((b >> 16) & jnp.uint32(1))
    b = b & jnp.uint32(0xFFFF0000)
    return jax.lax.bitcast_convert_type(b, jnp.float32)


def _knn_tc_body(tgt_ref, srcT_ref, idx_ref, w_ref):
    nrg = TB // RG

    def rg_loop(r, _):
        tx = tgt_ref[pl.ds(r * RG, RG), 0:1]
        ty = tgt_ref[pl.ds(r * RG, RG), 1:2]
        tz = tgt_ref[pl.ds(r * RG, RG), 2:3]
        t2 = (tx * tx + ty * ty) + tz * tz
        txb = _bf16_rne(tx)
        tyb = _bf16_rne(ty)
        tzb = _bf16_rne(tz)

        init = tuple(
            [jnp.full((RG, 128), BIGF)] * LVL
            + [jnp.zeros((RG, 128), jnp.int32)] * LVL
            + [jnp.full((RG, 128), BIGF)] * LVL
        )

        def q_loop(qq, carry):
            rv = list(carry[:LVL])
            rc = list(carry[LVL:2 * LVL])
            re = list(carry[2 * LVL:])
            sbase = qq * SLAB
            sx = srcT_ref[0:1, pl.ds(sbase, SLAB)]
            sy = srcT_ref[1:2, pl.ds(sbase, SLAB)]
            sz = srcT_ref[2:3, pl.ds(sbase, SLAB)]
            s2 = (sx * sx + sy * sy) + sz * sz
            sxb = _bf16_rne(sx)
            syb = _bf16_rne(sy)
            szb = _bf16_rne(sz)
            dotb = (txb * sxb + tyb * syb) + tzb * szb
            d2m = (t2 + s2) - 2.0 * dotb
            ax = tx - sx
            ay = ty - sy
            az = tz - sz
            d2e = (ax * ax + ay * ay) + az * az
            for c8 in range(SLAB // 128):
                x = lax.slice_in_dim(d2m, c8 * 128, (c8 + 1) * 128, axis=1)
                xe = lax.slice_in_dim(d2e, c8 * 128, (c8 + 1) * 128, axis=1)
                xc = jnp.full((RG, 128), qq * (SLAB // 128) + c8, jnp.int32)
                for l in range(LVL):
                    lt = x < rv[l]
                    nv = jnp.where(lt, x, rv[l])
                    nc = jnp.where(lt, xc, rc[l])
                    ne = jnp.where(lt, xe, re[l])
                    x = jnp.where(lt, rv[l], x)
                    xc = jnp.where(lt, rc[l], xc)
                    xe = jnp.where(lt, re[l], xe)
                    rv[l] = nv
                    rc[l] = nc
                    re[l] = ne
            return tuple(rv) + tuple(rc) + tuple(re)

        fin = lax.fori_loop(0, NQ, q_loop, init)
        rv = list(fin[:LVL])
        rc = list(fin[LVL:2 * LVL])
        re = list(fin[2 * LVL:])

        lane = lax.broadcasted_iota(jnp.int32, (RG, 128), 1)
        g = jnp.concatenate(rv, axis=1)
        li = jnp.concatenate([rc[l] * 128 + lane for l in range(LVL)], axis=1)
        ge = jnp.concatenate(re, axis=1)

        vals = []
        idxs = []
        for _ in range(KNN):
            m = jnp.min(g, axis=1, keepdims=True)
            cand = jnp.where(g == m, li, jnp.int32(2 * S))
            mi = jnp.min(cand, axis=1, keepdims=True)
            ev = jnp.min(jnp.where(cand == mi, ge, BIGF), axis=1,
                         keepdims=True)
            vals.append(ev)
            idxs.append(mi)
            g = jnp.where(cand == mi, BIGF, g)

        vmat = jnp.concatenate(vals, axis=1)
        imat = jnp.concatenate(idxs, axis=1)
        w = jnp.exp(-INVL2 * vmat)
        w = jnp.where(w < EPS, 0.0, w)
        wsum = jnp.sum(w, axis=1, keepdims=True) + EPS
        w = w / wsum
        imat_p = jnp.concatenate(
            [imat, jnp.zeros((RG, KP - KNN), jnp.int32)], axis=1)
        w_p = jnp.concatenate(
            [w, jnp.zeros((RG, WP - KNN), jnp.float32)], axis=1)
        idx_ref[pl.ds(r * RG, RG), :] = imat_p
        w_ref[pl.ds(r * RG, RG), :] = w_p
        return 0

    lax.fori_loop(0, nrg, rg_loop, 0)


def _knn_tc(target_position, srcT):
    return pl.pallas_call(
        _knn_tc_body,
        grid=(T // TB,),
        in_specs=[
            pl.BlockSpec((TB, 3), lambda i: (i, 0)),
            pl.BlockSpec((3, S), lambda i: (0, 0)),
        ],
        out_specs=[
            pl.BlockSpec((TB, KP), lambda i: (i, 0)),
            pl.BlockSpec((TB, WP), lambda i: (i, 0)),
        ],
        out_shape=[
            jax.ShapeDtypeStruct((T, KP), jnp.int32),
            jax.ShapeDtypeStruct((T, WP), jnp.float32),
        ],
    )(target_position, srcT)


NW = 32
TPW = T // NW
CH = 8
IDXC = CH * KP


@functools.lru_cache(maxsize=1)
def _sc_gather_build():
    @functools.partial(
        pl.kernel,
        mesh=plsc.VectorSubcoreMesh(core_axis_name="c", subcore_axis_name="s"),
        out_type=jax.ShapeDtypeStruct((T, D), jnp.float32),
        scratch_types=[
            pltpu.VMEM((IDXC,), jnp.int32),
            pltpu.VMEM((IDXC, D), jnp.float32),
            pltpu.VMEM((CH, WP), jnp.float32),
            pltpu.VMEM((CH, D), jnp.float32),
            pltpu.SemaphoreType.DMA,
        ],
    )
    def _sc_gather(idx_hbm, w_hbm, vals_hbm, out_hbm,
                   idx_v, rows_v, w_v, out_v, sem):
        wid = lax.axis_index("s") * 2 + lax.axis_index("c")
        base_t = wid * TPW

        def chunk(i, _):
            t0 = base_t + i * CH
            pltpu.sync_copy(idx_hbm.at[pl.ds(t0 * KP, IDXC)], idx_v)
            pltpu.sync_copy(w_hbm.at[pl.ds(t0, CH)], w_v)
            pltpu.async_copy(vals_hbm.at[idx_v], rows_v, sem).wait()
            for t in range(CH):
                wrow = w_v[t, :]
                accs = [jnp.zeros((16,), jnp.float32)
                        for _ in range(D // 16)]
                for j in range(KNN):
                    cidx = jnp.full((16,), j, jnp.int32)
                    wj = wrow.at[cidx].get(mode="promise_in_bounds")
                    rb = t * KP + j
                    for gseg in range(D // 16):
                        seg = rows_v[rb, pl.ds(gseg * 16, 16)]
                        accs[gseg] = accs[gseg] + wj * seg
                for gseg in range(D // 16):
                    out_v[t, pl.ds(gseg * 16, 16)] = accs[gseg]
            pltpu.sync_copy(out_v, out_hbm.at[pl.ds(t0, CH)])
            return 0

        lax.fori_loop(0, TPW // CH, chunk, 0)

    return _sc_gather


def kernel(source_position, target_position, source_values):
    srcT = source_position.T
    idx, w = _knn_tc(target_position, srcT)
    out = _sc_gather_build()(idx.reshape(-1), w, source_values)
    return out


# speedup vs baseline: 1.6483x; 1.2197x over previous
import functools

import jax
import jax.numpy as jnp
from jax import lax
from jax.experimental import pallas as pl
from jax.experimental.pallas import tpu as pltpu
from jax.experimental.pallas import tpu_sc as plsc

T = 16384
S = 16384
D = 128
KNN = 9
LVL = 4
TB = 256
RG = 8
LW = 256
SLAB = 512
NQ = S // SLAB
GW = LVL * LW
KP = 12
WP = 16
INVL2 = 100.0
EPS = 1e-5
BIGF = 3.0e38


def _bf16_rne(x):
    b = jax.lax.bitcast_convert_type(x, jnp.uint32)
    b = b + jnp.uint32(0x7FFF) + ((b >> 16) & jnp.uint32(1))
    b = b & jnp.uint32(0xFFFF0000)
    return jax.lax.bitcast_convert_type(b, jnp.float32)


def _knn_tc_body(tgt_ref, srcT_ref, idx_ref, w_ref, gv_ref, gc_ref, ge_ref):
    nrg = TB // RG

    def rg_loop(r, _):
        tx = tgt_ref[pl.ds(r * RG, RG), 0:1]
        ty = tgt_ref[pl.ds(r * RG, RG), 1:2]
        tz = tgt_ref[pl.ds(r * RG, RG), 2:3]
        t2 = (tx * tx + ty * ty) + tz * tz
        txb = _bf16_rne(tx)
        tyb = _bf16_rne(ty)
        tzb = _bf16_rne(tz)

        init = tuple(
            [jnp.full((RG, LW), BIGF)] * LVL
            + [jnp.zeros((RG, LW), jnp.int32)] * LVL
            + [jnp.full((RG, LW), BIGF)] * LVL
        )

        def q_loop(qq, carry):
            rv = list(carry[:LVL])
            rc = list(carry[LVL:2 * LVL])
            re = list(carry[2 * LVL:])
            sbase = qq * SLAB
            sx = srcT_ref[0:1, pl.ds(sbase, SLAB)]
            sy = srcT_ref[1:2, pl.ds(sbase, SLAB)]
            sz = srcT_ref[2:3, pl.ds(sbase, SLAB)]
            s2 = (sx * sx + sy * sy) + sz * sz
            sxb = _bf16_rne(sx)
            syb = _bf16_rne(sy)
            szb = _bf16_rne(sz)
            dotb = (txb * sxb + tyb * syb) + tzb * szb
            d2m = (t2 + s2) - 2.0 * dotb
            ax = tx - sx
            ay = ty - sy
            az = tz - sz
            d2e = (ax * ax + ay * ay) + az * az
            for cs in range(SLAB // LW):
                x = lax.slice_in_dim(d2m, cs * LW, (cs + 1) * LW, axis=1)
                xe = lax.slice_in_dim(d2e, cs * LW, (cs + 1) * LW, axis=1)
                xc = jnp.full((RG, LW), qq * (SLAB // LW) + cs, jnp.int32)
                for l in range(LVL):
                    lt = x < rv[l]
                    nv = jnp.where(lt, x, rv[l])
                    nc = jnp.where(lt, xc, rc[l])
                    ne = jnp.where(lt, xe, re[l])
                    x = jnp.where(lt, rv[l], x)
                    xc = jnp.where(lt, rc[l], xc)
                    xe = jnp.where(lt, re[l], xe)
                    rv[l] = nv
                    rc[l] = nc
                    re[l] = ne
            return tuple(rv) + tuple(rc) + tuple(re)

        fin = lax.fori_loop(0, NQ, q_loop, init)
        for l in range(LVL):
            gv_ref[pl.ds(r * RG, RG), pl.ds(l * LW, LW)] = fin[l]
            gc_ref[pl.ds(r * RG, RG), pl.ds(l * LW, LW)] = fin[LVL + l]
            ge_ref[pl.ds(r * RG, RG), pl.ds(l * LW, LW)] = fin[2 * LVL + l]
        return 0

    lax.fori_loop(0, nrg, rg_loop, 0)

    g = gv_ref[...]
    lane = lax.broadcasted_iota(jnp.int32, (TB, GW), 1)
    lane = lax.rem(lane, jnp.int32(LW))
    li = gc_ref[...] * LW + lane
    ge = ge_ref[...]

    vals = []
    idxs = []
    for _ in range(KNN):
        m = jnp.min(g, axis=1, keepdims=True)
        cand = jnp.where(g == m, li, jnp.int32(2 * S))
        mi = jnp.min(cand, axis=1, keepdims=True)
        ev = jnp.min(jnp.where(cand == mi, ge, BIGF), axis=1,
                     keepdims=True)
        vals.append(ev)
        idxs.append(mi)
        g = jnp.where(cand == mi, BIGF, g)

    vmat = jnp.concatenate(vals, axis=1)
    imat = jnp.concatenate(idxs, axis=1)
    w = jnp.exp(-INVL2 * vmat)
    w = jnp.where(w < EPS, 0.0, w)
    wsum = jnp.sum(w, axis=1, keepdims=True) + EPS
    w = w / wsum
    idx_ref[...] = jnp.concatenate(
        [imat, jnp.zeros((TB, KP - KNN), jnp.int32)], axis=1)
    w_ref[...] = jnp.concatenate(
        [w, jnp.zeros((TB, WP - KNN), jnp.float32)], axis=1)


def _knn_tc(target_position, srcT):
    return pl.pallas_call(
        _knn_tc_body,
        grid=(T // TB,),
        in_specs=[
            pl.BlockSpec((TB, 3), lambda i: (i, 0)),
            pl.BlockSpec((3, S), lambda i: (0, 0)),
        ],
        out_specs=[
            pl.BlockSpec((TB, KP), lambda i: (i, 0)),
            pl.BlockSpec((TB, WP), lambda i: (i, 0)),
        ],
        out_shape=[
            jax.ShapeDtypeStruct((T, KP), jnp.int32),
            jax.ShapeDtypeStruct((T, WP), jnp.float32),
        ],
        scratch_shapes=[
            pltpu.VMEM((TB, GW), jnp.float32),
            pltpu.VMEM((TB, GW), jnp.int32),
            pltpu.VMEM((TB, GW), jnp.float32),
        ],
    )(target_position, srcT)


NW = 32
TPW = T // NW
CH = 8
IDXC = CH * KP


@functools.lru_cache(maxsize=1)
def _sc_gather_build():
    @functools.partial(
        pl.kernel,
        mesh=plsc.VectorSubcoreMesh(core_axis_name="c", subcore_axis_name="s"),
        out_type=jax.ShapeDtypeStruct((T, D), jnp.float32),
        scratch_types=[
            pltpu.VMEM((IDXC,), jnp.int32),
            pltpu.VMEM((IDXC, D), jnp.float32),
            pltpu.VMEM((CH, WP), jnp.float32),
            pltpu.VMEM((CH, D), jnp.float32),
            pltpu.SemaphoreType.DMA,
        ],
    )
    def _sc_gather(idx_hbm, w_hbm, vals_hbm, out_hbm,
                   idx_v, rows_v, w_v, out_v, sem):
        wid = lax.axis_index("s") * 2 + lax.axis_index("c")
        base_t = wid * TPW

        def chunk(i, _):
            t0 = base_t + i * CH
            pltpu.sync_copy(idx_hbm.at[pl.ds(t0 * KP, IDXC)], idx_v)
            pltpu.sync_copy(w_hbm.at[pl.ds(t0, CH)], w_v)
            pltpu.async_copy(vals_hbm.at[idx_v], rows_v, sem).wait()
            for t in range(CH):
                wrow = w_v[t, :]
                accs = [jnp.zeros((16,), jnp.float32)
                        for _ in range(D // 16)]
                for j in range(KNN):
                    cidx = jnp.full((16,), j, jnp.int32)
                    wj = wrow.at[cidx].get(mode="promise_in_bounds")
                    rb = t * KP + j
                    for gseg in range(D // 16):
                        seg = rows_v[rb, pl.ds(gseg * 16, 16)]
                        accs[gseg] = accs[gseg] + wj * seg
                for gseg in range(D // 16):
                    out_v[t, pl.ds(gseg * 16, 16)] = accs[gseg]
            pltpu.sync_copy(out_v, out_hbm.at[pl.ds(t0, CH)])
            return 0

        lax.fori_loop(0, TPW // CH, chunk, 0)

    return _sc_gather


def kernel(source_position, target_position, source_values):
    srcT = source_position.T
    idx, w = _knn_tc(target_position, srcT)
    out = _sc_gather_build()(idx.reshape(-1), w, source_values)
    return out
